# Initial kernel scaffold; baseline (speedup 1.0000x reference)
import functools

import jax
import jax.numpy as jnp
from jax import lax
from jax.experimental import pallas as pl
from jax.experimental.pallas import tpu as pltpu
from jax.experimental.pallas import tpu_sc as plsc

N_FEATURES = 100
N_QUANTILES = 1000
EMB_DIM = 32
N_EMB = (N_QUANTILES + 2) * N_FEATURES

BATCH = 16384
TOTAL = BATCH * N_FEATURES
NW = 32
PER_W = TOTAL // NW
G = 128
NG = 4
CHUNK = G * NG
N_CHUNKS = PER_W // CHUNK
LANES = 16


def _body(x_hbm, m_hbm, off_hbm, emb_hbm, out_hbm,
          x_v, m_v, off_v, ilo_v, ihi_v, wl_v, wh_v, lo_v, hi_v, gsem):
    wid = lax.axis_index("s") * 2 + lax.axis_index("c")
    th = jnp.float32(1e-06)
    one = jnp.float32(1.0)
    nq = jnp.float32(N_QUANTILES)
    iota = lax.iota(jnp.int32, LANES)

    pltpu.sync_copy(off_hbm, off_v)

    def chunk_body(c, carry):
        base = wid * PER_W + c * CHUNK
        pltpu.sync_copy(x_hbm.at[pl.ds(base, CHUNK)], x_v)
        pltpu.sync_copy(m_hbm.at[pl.ds(base, CHUNK)], m_v)

        for i in range(CHUNK // LANES):
            sl = pl.ds(i * LANES, LANES)
            pos = iota + (base + i * LANES)
            feat = lax.rem(pos, jnp.int32(N_FEATURES))
            off = plsc.load_gather(off_v, [feat])
            xc = jnp.minimum(jnp.maximum(x_v[sl], th), one - th)
            y = xc * nq
            ili = y.astype(jnp.int32)
            ihi = (y + one).astype(jnp.int32)
            xl = ili.astype(jnp.float32) / nq
            xh = ihi.astype(jnp.float32) / nq
            wl_v[sl] = (xh - xc) * nq
            wh_v[sl] = (xc - xl) * nq
            m = m_v[sl]
            ilo_v[i // 8, pl.ds((i % 8) * LANES, LANES)] = (ili + 1) * m + off
            ihi_v[i // 8, pl.ds((i % 8) * LANES, LANES)] = (ihi + 1) * m + off

        handles = []
        for j in range(NG):
            handles.append(pltpu.async_copy(
                emb_hbm.at[ilo_v.at[j]], lo_v.at[pl.ds(j * G, G)], gsem))
            handles.append(pltpu.async_copy(
                emb_hbm.at[ihi_v.at[j]], hi_v.at[pl.ds(j * G, G)], gsem))
        for h in handles:
            h.wait()

        def lerp_group(q, inner):
            r = iota + q * LANES
            wl = wl_v[pl.ds(q * LANES, LANES)]
            wh = wh_v[pl.ds(q * LANES, LANES)]
            for d in range(EMB_DIM):
                dv = jnp.full((LANES,), d, jnp.int32)
                lo = plsc.load_gather(lo_v, [r, dv])
                hi = plsc.load_gather(hi_v, [r, dv])
                plsc.store_scatter(lo_v, [r, dv], lo * wl + hi * wh)
            return inner

        lax.fori_loop(0, CHUNK // LANES, lerp_group, 0)

        pltpu.sync_copy(lo_v, out_hbm.at[pl.ds(base, CHUNK)])
        return carry

    lax.fori_loop(0, N_CHUNKS, chunk_body, 0)


@jax.jit
def kernel(x, mask, rand_table, emb):
    x_flat = x.reshape(TOTAL)
    m_flat = mask.reshape(TOTAL)
    table_shift = jnp.int32(N_EMB) * jnp.asarray(rand_table, jnp.int32)
    off = (N_QUANTILES + 2) * jnp.arange(N_FEATURES, dtype=jnp.int32) + table_shift
    off_pad = jnp.zeros((128,), jnp.int32).at[:N_FEATURES].set(off)

    mesh = plsc.VectorSubcoreMesh(core_axis_name="c", subcore_axis_name="s")
    run = pl.kernel(
        _body,
        out_type=jax.ShapeDtypeStruct((TOTAL, EMB_DIM), jnp.float32),
        mesh=mesh,
        scratch_types=[
            pltpu.VMEM((CHUNK,), jnp.float32),
            pltpu.VMEM((CHUNK,), jnp.int32),
            pltpu.VMEM((128,), jnp.int32),
            pltpu.VMEM((NG, G), jnp.int32),
            pltpu.VMEM((NG, G), jnp.int32),
            pltpu.VMEM((CHUNK,), jnp.float32),
            pltpu.VMEM((CHUNK,), jnp.float32),
            pltpu.VMEM((CHUNK, EMB_DIM), jnp.float32),
            pltpu.VMEM((CHUNK, EMB_DIM), jnp.float32),
            pltpu.SemaphoreType.DMA,
        ],
    )
    out = run(x_flat, m_flat, off_pad, emb)
    return out.reshape(BATCH, N_FEATURES, EMB_DIM)


# speedup vs baseline: 2.4117x; 2.4117x over previous
import functools

import jax
import jax.numpy as jnp
from jax import lax
from jax.experimental import pallas as pl
from jax.experimental.pallas import tpu as pltpu
from jax.experimental.pallas import tpu_sc as plsc

N_FEATURES = 100
N_QUANTILES = 1000
EMB_DIM = 32
N_EMB = (N_QUANTILES + 2) * N_FEATURES

BATCH = 16384
TOTAL = BATCH * N_FEATURES
NW = 32
PER_W = TOTAL // NW
G = 128
NG = 4
CHUNK = G * NG
N_CHUNKS = PER_W // CHUNK
LANES = 16


def _body(x_hbm, m_hbm, off_hbm, emb_hbm, out_hbm,
          x_v, m_v, off_v, ilo_v, ihi_v, wl_v, wh_v, lo_v, hi_v, gsem):
    wid = lax.axis_index("s") * 2 + lax.axis_index("c")
    th = jnp.float32(1e-06)
    one = jnp.float32(1.0)
    nq = jnp.float32(N_QUANTILES)
    iota = lax.iota(jnp.int32, LANES)

    pltpu.sync_copy(off_hbm, off_v)

    def chunk_body(c, carry):
        base = wid * PER_W + c * CHUNK
        pltpu.sync_copy(x_hbm.at[pl.ds(base, CHUNK)], x_v)
        pltpu.sync_copy(m_hbm.at[pl.ds(base, CHUNK)], m_v)

        for i in range(CHUNK // LANES):
            sl = pl.ds(i * LANES, LANES)
            pos = iota + (base + i * LANES)
            feat = lax.rem(pos, jnp.int32(N_FEATURES))
            off = plsc.load_gather(off_v, [feat])
            xc = jnp.minimum(jnp.maximum(x_v[sl], th), one - th)
            y = xc * nq
            ili = y.astype(jnp.int32)
            ihi = (y + one).astype(jnp.int32)
            xl = ili.astype(jnp.float32) / nq
            xh = ihi.astype(jnp.float32) / nq
            wl_v[sl] = (xh - xc) * nq
            wh_v[sl] = (xc - xl) * nq
            m = m_v[sl]
            ilo_v[i // 8, pl.ds((i % 8) * LANES, LANES)] = (ili + 1) * m + off
            ihi_v[i // 8, pl.ds((i % 8) * LANES, LANES)] = (ihi + 1) * m + off

        handles = []
        for j in range(NG):
            handles.append(pltpu.async_copy(
                emb_hbm.at[ilo_v.at[j]], lo_v.at[pl.ds(j * G, G)], gsem))
            handles.append(pltpu.async_copy(
                emb_hbm.at[ihi_v.at[j]], hi_v.at[pl.ds(j * G, G)], gsem))
        for h in handles:
            h.wait()

        def lerp_group(q, inner):
            r = iota + q * LANES
            wl = wl_v[pl.ds(q * LANES, LANES)]
            wh = wh_v[pl.ds(q * LANES, LANES)]
            for d in range(EMB_DIM):
                dv = jnp.full((LANES,), d, jnp.int32)
                lo = plsc.load_gather(lo_v, [r, dv])
                hi = plsc.load_gather(hi_v, [r, dv])
                plsc.store_scatter(lo_v, [r, dv], lo * wl + hi * wh)
            return inner

        lax.fori_loop(0, CHUNK // LANES, lerp_group, 0)

        pltpu.sync_copy(lo_v, out_hbm.at[pl.ds(base, CHUNK)])
        return carry

    lax.fori_loop(0, N_CHUNKS, chunk_body, 0)


@jax.jit
def kernel(x, mask, rand_table, emb):
    x_flat = x.reshape(TOTAL)
    m_flat = mask.reshape(TOTAL)
    table_shift = jnp.int32(N_EMB) * jnp.asarray(rand_table, jnp.int32)
    off = (N_QUANTILES + 2) * jnp.arange(N_FEATURES, dtype=jnp.int32) + table_shift
    off_pad = jnp.zeros((128,), jnp.int32).at[:N_FEATURES].set(off)

    mesh = plsc.VectorSubcoreMesh(core_axis_name="c", subcore_axis_name="s")
    run = pl.kernel(
        _body,
        out_type=jax.ShapeDtypeStruct((TOTAL, EMB_DIM), jnp.float32),
        mesh=mesh,
        compiler_params=pltpu.CompilerParams(needs_layout_passes=False,
                                              use_tc_tiling_on_sc=False),
        scratch_types=[
            pltpu.VMEM((CHUNK,), jnp.float32),
            pltpu.VMEM((CHUNK,), jnp.int32),
            pltpu.VMEM((128,), jnp.int32),
            pltpu.VMEM((NG, G), jnp.int32),
            pltpu.VMEM((NG, G), jnp.int32),
            pltpu.VMEM((CHUNK,), jnp.float32),
            pltpu.VMEM((CHUNK,), jnp.float32),
            pltpu.VMEM((CHUNK, EMB_DIM), jnp.float32),
            pltpu.VMEM((CHUNK, EMB_DIM), jnp.float32),
            pltpu.SemaphoreType.DMA,
        ],
    )
    out = run(x_flat, m_flat, off_pad, emb)
    return out.reshape(BATCH, N_FEATURES, EMB_DIM)


# speedup vs baseline: 4.9186x; 2.0395x over previous
import functools

import jax
import jax.numpy as jnp
from jax import lax
from jax.experimental import pallas as pl
from jax.experimental.pallas import tpu as pltpu
from jax.experimental.pallas import tpu_sc as plsc

N_FEATURES = 100
N_QUANTILES = 1000
EMB_DIM = 32
N_EMB = (N_QUANTILES + 2) * N_FEATURES

BATCH = 16384
NW = 32
ROWS_W = BATCH // NW
R = 8
CHUNK = R * N_FEATURES
N_CHUNKS = ROWS_W // R
LANES = 16


def _body(x_hbm, m_hbm, off_hbm, emb_hbm, out_hbm,
          x_v, m_v, off_v, ilo_v, ihi_v, wl_v, wh_v, lo_v, hi_v, out_v, gsem):
    wid = lax.axis_index("s") * 2 + lax.axis_index("c")
    th = jnp.float32(1e-06)
    one = jnp.float32(1.0)
    nq = jnp.float32(N_QUANTILES)
    nf = jnp.int32(N_FEATURES)
    iota = lax.iota(jnp.int32, LANES)

    pltpu.sync_copy(off_hbm, off_v)

    def chunk_body(c, carry):
        row0 = wid * ROWS_W + c * R
        pltpu.sync_copy(x_hbm.at[pl.ds(row0, R)], x_v)
        pltpu.sync_copy(m_hbm.at[pl.ds(row0, R)], m_v)

        def idx_step(i, inner):
            lin = iota + i * LANES
            r = lin // nf
            f = lax.rem(lin, nf)
            off = plsc.load_gather(off_v, [f])
            xv = plsc.load_gather(x_v, [r, f])
            m = plsc.load_gather(m_v, [r, f])
            xc = jnp.minimum(jnp.maximum(xv, th), one - th)
            y = xc * nq
            ili = y.astype(jnp.int32)
            ihi = (y + one).astype(jnp.int32)
            xl = ili.astype(jnp.float32) / nq
            xh = ihi.astype(jnp.float32) / nq
            sl = pl.ds(i * LANES, LANES)
            wl_v[sl] = (xh - xc) * nq
            wh_v[sl] = (xc - xl) * nq
            plsc.store_scatter(ilo_v, [r, f], (ili + 1) * m + off)
            plsc.store_scatter(ihi_v, [r, f], (ihi + 1) * m + off)
            return inner

        lax.fori_loop(0, CHUNK // LANES, idx_step, 0)

        handles = []
        for j in range(R):
            handles.append(pltpu.async_copy(
                emb_hbm.at[ilo_v.at[j]], lo_v.at[pl.ds(j * N_FEATURES, N_FEATURES)],
                gsem))
            handles.append(pltpu.async_copy(
                emb_hbm.at[ihi_v.at[j]], hi_v.at[pl.ds(j * N_FEATURES, N_FEATURES)],
                gsem))
        for h in handles:
            h.wait()

        def lerp_group(q, inner):
            lin = iota + q * LANES
            r = lin // nf
            f = lax.rem(lin, nf)
            wl = wl_v[pl.ds(q * LANES, LANES)]
            wh = wh_v[pl.ds(q * LANES, LANES)]
            for d in range(EMB_DIM):
                dv = jnp.full((LANES,), d, jnp.int32)
                lo = plsc.load_gather(lo_v, [lin, dv])
                hi = plsc.load_gather(hi_v, [lin, dv])
                plsc.store_scatter(out_v, [r, f, dv], lo * wl + hi * wh)
            return inner

        lax.fori_loop(0, CHUNK // LANES, lerp_group, 0)

        pltpu.sync_copy(out_v, out_hbm.at[pl.ds(row0, R)])
        return carry

    lax.fori_loop(0, N_CHUNKS, chunk_body, 0)


@jax.jit
def kernel(x, mask, rand_table, emb):
    table_shift = jnp.int32(N_EMB) * jnp.asarray(rand_table, jnp.int32)
    off = (N_QUANTILES + 2) * jnp.arange(N_FEATURES, dtype=jnp.int32) + table_shift
    off_pad = jnp.zeros((128,), jnp.int32).at[:N_FEATURES].set(off)

    mesh = plsc.VectorSubcoreMesh(core_axis_name="c", subcore_axis_name="s")
    run = pl.kernel(
        _body,
        out_type=jax.ShapeDtypeStruct((BATCH, N_FEATURES, EMB_DIM), jnp.float32),
        mesh=mesh,
        compiler_params=pltpu.CompilerParams(needs_layout_passes=False,
                                             use_tc_tiling_on_sc=False),
        scratch_types=[
            pltpu.VMEM((R, N_FEATURES), jnp.float32),
            pltpu.VMEM((R, N_FEATURES), jnp.int32),
            pltpu.VMEM((128,), jnp.int32),
            pltpu.VMEM((R, N_FEATURES), jnp.int32),
            pltpu.VMEM((R, N_FEATURES), jnp.int32),
            pltpu.VMEM((CHUNK,), jnp.float32),
            pltpu.VMEM((CHUNK,), jnp.float32),
            pltpu.VMEM((CHUNK, EMB_DIM), jnp.float32),
            pltpu.VMEM((CHUNK, EMB_DIM), jnp.float32),
            pltpu.VMEM((R, N_FEATURES, EMB_DIM), jnp.float32),
            pltpu.SemaphoreType.DMA,
        ],
    )
    return run(x, mask, off_pad, emb)


# speedup vs baseline: 10.4841x; 2.1315x over previous
import functools

import jax
import jax.numpy as jnp
from jax import lax
from jax.experimental import pallas as pl
from jax.experimental.pallas import tpu as pltpu
from jax.experimental.pallas import tpu_sc as plsc

N_FEATURES = 100
N_QUANTILES = 1000
EMB_DIM = 32
N_EMB = (N_QUANTILES + 2) * N_FEATURES

BATCH = 16384
NW = 32
ROWS_W = BATCH // NW
R = 8
CHUNK = R * N_FEATURES
N_CHUNKS = ROWS_W // R
LANES = 16


def _body(x_hbm, m_hbm, off_hbm, emb_hbm, out_hbm,
          x_v, m_v, off_v, ilo_v, ihi_v, wl_v, wh_v, lo_v, hi_v, out_v, gsem):
    wid = lax.axis_index("s") * 2 + lax.axis_index("c")
    th = jnp.float32(1e-06)
    one = jnp.float32(1.0)
    nq = jnp.float32(N_QUANTILES)
    nf = jnp.int32(N_FEATURES)
    iota = lax.iota(jnp.int32, LANES)

    pltpu.sync_copy(off_hbm, off_v)

    def chunk_body(c, carry):
        row0 = wid * ROWS_W + c * R
        pltpu.sync_copy(x_hbm.at[pl.ds(row0, R)], x_v)
        pltpu.sync_copy(m_hbm.at[pl.ds(row0, R)], m_v)

        def idx_step(i, inner):
            lin = iota + i * LANES
            r = lin // nf
            f = lax.rem(lin, nf)
            off = plsc.load_gather(off_v, [f])
            xv = plsc.load_gather(x_v, [r, f])
            m = plsc.load_gather(m_v, [r, f])
            xc = jnp.minimum(jnp.maximum(xv, th), one - th)
            y = xc * nq
            ili = y.astype(jnp.int32)
            ihi = (y + one).astype(jnp.int32)
            xl = ili.astype(jnp.float32) / nq
            xh = ihi.astype(jnp.float32) / nq
            sl = pl.ds(i * LANES, LANES)
            wl_v[sl] = (xh - xc) * nq
            wh_v[sl] = (xc - xl) * nq
            plsc.store_scatter(ilo_v, [r, f], (ili + 1) * m + off)
            plsc.store_scatter(ihi_v, [r, f], (ihi + 1) * m + off)
            return inner

        lax.fori_loop(0, CHUNK // LANES, idx_step, 0)

        handles = []
        for j in range(R):
            handles.append(pltpu.async_copy(
                emb_hbm.at[ilo_v.at[j]], lo_v.at[pl.ds(j * N_FEATURES, N_FEATURES)],
                gsem))
            handles.append(pltpu.async_copy(
                emb_hbm.at[ihi_v.at[j]], hi_v.at[pl.ds(j * N_FEATURES, N_FEATURES)],
                gsem))
        for h in handles:
            h.wait()

        def lerp_group(q, inner):
            lin = iota + q * LANES
            r = lin // nf
            f = lax.rem(lin, nf)
            wl = wl_v[pl.ds(q * LANES, LANES)]
            wh = wh_v[pl.ds(q * LANES, LANES)]
            for d in range(EMB_DIM):
                dv = (iota + d) & (EMB_DIM - 1)
                lo = plsc.load_gather(lo_v, [lin, dv])
                hi = plsc.load_gather(hi_v, [lin, dv])
                plsc.store_scatter(out_v, [r, f, dv], lo * wl + hi * wh)
            return inner

        lax.fori_loop(0, CHUNK // LANES, lerp_group, 0)

        pltpu.sync_copy(out_v, out_hbm.at[pl.ds(row0, R)])
        return carry

    lax.fori_loop(0, N_CHUNKS, chunk_body, 0)


@jax.jit
def kernel(x, mask, rand_table, emb):
    table_shift = jnp.int32(N_EMB) * jnp.asarray(rand_table, jnp.int32)
    off = (N_QUANTILES + 2) * jnp.arange(N_FEATURES, dtype=jnp.int32) + table_shift
    off_pad = jnp.zeros((128,), jnp.int32).at[:N_FEATURES].set(off)

    mesh = plsc.VectorSubcoreMesh(core_axis_name="c", subcore_axis_name="s")
    run = pl.kernel(
        _body,
        out_type=jax.ShapeDtypeStruct((BATCH, N_FEATURES, EMB_DIM), jnp.float32),
        mesh=mesh,
        compiler_params=pltpu.CompilerParams(needs_layout_passes=False,
                                             use_tc_tiling_on_sc=False),
        scratch_types=[
            pltpu.VMEM((R, N_FEATURES), jnp.float32),
            pltpu.VMEM((R, N_FEATURES), jnp.int32),
            pltpu.VMEM((128,), jnp.int32),
            pltpu.VMEM((R, N_FEATURES), jnp.int32),
            pltpu.VMEM((R, N_FEATURES), jnp.int32),
            pltpu.VMEM((CHUNK,), jnp.float32),
            pltpu.VMEM((CHUNK,), jnp.float32),
            pltpu.VMEM((CHUNK, EMB_DIM), jnp.float32),
            pltpu.VMEM((CHUNK, EMB_DIM), jnp.float32),
            pltpu.VMEM((R, N_FEATURES, EMB_DIM), jnp.float32),
            pltpu.SemaphoreType.DMA,
        ],
    )
    return run(x, mask, off_pad, emb)


# speedup vs baseline: 12.7581x; 1.2169x over previous
import functools

import jax
import jax.numpy as jnp
from jax import lax
from jax.experimental import pallas as pl
from jax.experimental.pallas import tpu as pltpu
from jax.experimental.pallas import tpu_sc as plsc

N_FEATURES = 100
N_QUANTILES = 1000
EMB_DIM = 32
N_EMB = (N_QUANTILES + 2) * N_FEATURES

BATCH = 16384
NW = 32
ROWS_W = BATCH // NW
R = 8
CHUNK = R * N_FEATURES
N_CHUNKS = ROWS_W // R
LANES = 16


def _body(x_hbm, m_hbm, off_hbm, emb_hbm, out_hbm,
          x_v, m_v, off_v, ilo_v, ihi_v, wl_v, wh_v, lo_v, hi_v, out_v, gsem):
    wid = lax.axis_index("s") * 2 + lax.axis_index("c")
    th = jnp.float32(1e-06)
    one = jnp.float32(1.0)
    nq = jnp.float32(N_QUANTILES)
    nf = jnp.int32(N_FEATURES)
    iota = lax.iota(jnp.int32, LANES)

    pltpu.sync_copy(off_hbm, off_v)

    def chunk_body(c, carry):
        row0 = wid * ROWS_W + c * R
        pltpu.sync_copy(x_hbm.at[pl.ds(row0, R)], x_v)
        pltpu.sync_copy(m_hbm.at[pl.ds(row0, R)], m_v)

        @plsc.parallel_loop(0, CHUNK // LANES, unroll=2)
        def idx_step(i):
            lin = iota + i * LANES
            r = lin // nf
            f = lax.rem(lin, nf)
            off = plsc.load_gather(off_v, [f])
            xv = plsc.load_gather(x_v, [r, f])
            m = plsc.load_gather(m_v, [r, f])
            xc = jnp.minimum(jnp.maximum(xv, th), one - th)
            y = xc * nq
            ili = y.astype(jnp.int32)
            ihi = (y + one).astype(jnp.int32)
            xl = ili.astype(jnp.float32) / nq
            xh = ihi.astype(jnp.float32) / nq
            sl = pl.ds(i * LANES, LANES)
            wl_v[sl] = (xh - xc) * nq
            wh_v[sl] = (xc - xl) * nq
            plsc.store_scatter(ilo_v, [r, f], (ili + 1) * m + off)
            plsc.store_scatter(ihi_v, [r, f], (ihi + 1) * m + off)

        handles = []
        for j in range(R):
            handles.append(pltpu.async_copy(
                emb_hbm.at[ilo_v.at[j]], lo_v.at[pl.ds(j * N_FEATURES, N_FEATURES)],
                gsem))
            handles.append(pltpu.async_copy(
                emb_hbm.at[ihi_v.at[j]], hi_v.at[pl.ds(j * N_FEATURES, N_FEATURES)],
                gsem))
        for h in handles:
            h.wait()

        @plsc.parallel_loop(0, CHUNK // LANES, unroll=2)
        def lerp_group(q):
            lin = iota + q * LANES
            r = lin // nf
            f = lax.rem(lin, nf)
            wl = wl_v[pl.ds(q * LANES, LANES)]
            wh = wh_v[pl.ds(q * LANES, LANES)]
            for d in range(EMB_DIM):
                dv = (iota + d) & (EMB_DIM - 1)
                lo = plsc.load_gather(lo_v, [lin, dv])
                hi = plsc.load_gather(hi_v, [lin, dv])
                plsc.store_scatter(out_v, [r, f, dv], lo * wl + hi * wh)

        pltpu.sync_copy(out_v, out_hbm.at[pl.ds(row0, R)])
        return carry

    lax.fori_loop(0, N_CHUNKS, chunk_body, 0)


@jax.jit
def kernel(x, mask, rand_table, emb):
    table_shift = jnp.int32(N_EMB) * jnp.asarray(rand_table, jnp.int32)
    off = (N_QUANTILES + 2) * jnp.arange(N_FEATURES, dtype=jnp.int32) + table_shift
    off_pad = jnp.zeros((128,), jnp.int32).at[:N_FEATURES].set(off)

    mesh = plsc.VectorSubcoreMesh(core_axis_name="c", subcore_axis_name="s")
    run = pl.kernel(
        _body,
        out_type=jax.ShapeDtypeStruct((BATCH, N_FEATURES, EMB_DIM), jnp.float32),
        mesh=mesh,
        compiler_params=pltpu.CompilerParams(needs_layout_passes=False,
                                             use_tc_tiling_on_sc=False),
        scratch_types=[
            pltpu.VMEM((R, N_FEATURES), jnp.float32),
            pltpu.VMEM((R, N_FEATURES), jnp.int32),
            pltpu.VMEM((128,), jnp.int32),
            pltpu.VMEM((R, N_FEATURES), jnp.int32),
            pltpu.VMEM((R, N_FEATURES), jnp.int32),
            pltpu.VMEM((CHUNK,), jnp.float32),
            pltpu.VMEM((CHUNK,), jnp.float32),
            pltpu.VMEM((CHUNK, EMB_DIM), jnp.float32),
            pltpu.VMEM((CHUNK, EMB_DIM), jnp.float32),
            pltpu.VMEM((R, N_FEATURES, EMB_DIM), jnp.float32),
            pltpu.SemaphoreType.DMA,
        ],
    )
    return run(x, mask, off_pad, emb)


# speedup vs baseline: 15.1596x; 1.1882x over previous
import functools

import jax
import jax.numpy as jnp
from jax import lax
from jax.experimental import pallas as pl
from jax.experimental.pallas import tpu as pltpu
from jax.experimental.pallas import tpu_sc as plsc

N_FEATURES = 100
N_QUANTILES = 1000
EMB_DIM = 32
N_EMB = (N_QUANTILES + 2) * N_FEATURES

BATCH = 16384
NW = 32
ROWS_W = BATCH // NW
BB = 128
NBB = ROWS_W // BB
FG = 4
NFG = N_FEATURES // FG
CHUNK = FG * BB
LANES = 16
DT = EMB_DIM // 8


def _body(x_hbm, m_hbm, off_hbm, emb_hbm, out_hbm,
          x_v, m_v, off_v, ilo_v, ihi_v, wl_v, wh_v, lo_v, hi_v, out_v, gsem):
    wid = lax.axis_index("s") * 2 + lax.axis_index("c")
    th = jnp.float32(1e-06)
    one = jnp.float32(1.0)
    nq = jnp.float32(N_QUANTILES)
    iota = lax.iota(jnp.int32, LANES)

    pltpu.sync_copy(off_hbm, off_v)

    def bb_body(blk, carry):
        b0 = wid * ROWS_W + blk * BB
        pltpu.sync_copy(x_hbm.at[pl.ds(b0, BB)], x_v)
        pltpu.sync_copy(m_hbm.at[pl.ds(b0, BB)], m_v)

        def fg_body(g, inner):
            f0 = g * FG

            @plsc.parallel_loop(0, CHUNK // LANES, unroll=2)
            def idx_step(i):
                lin = iota + i * LANES
                fl = lin >> 7
                b = lin & (BB - 1)
                f = f0 + fl
                off = plsc.load_gather(off_v, [f])
                xv = plsc.load_gather(x_v, [b, f])
                m = plsc.load_gather(m_v, [b, f])
                xc = jnp.minimum(jnp.maximum(xv, th), one - th)
                y = xc * nq
                ili = y.astype(jnp.int32)
                ihi = (y + one).astype(jnp.int32)
                xl = ili.astype(jnp.float32) / nq
                xh = ihi.astype(jnp.float32) / nq
                sl = pl.ds(i * LANES, LANES)
                wl_v[sl] = (xh - xc) * nq
                wh_v[sl] = (xc - xl) * nq
                plsc.store_scatter(ilo_v, [fl, b], (ili + 1) * m + off)
                plsc.store_scatter(ihi_v, [fl, b], (ihi + 1) * m + off)

            handles = []
            for j in range(FG):
                handles.append(pltpu.async_copy(
                    emb_hbm.at[ilo_v.at[j]],
                    lo_v.at[pl.ds(j * BB, BB)], gsem))
                handles.append(pltpu.async_copy(
                    emb_hbm.at[ihi_v.at[j]],
                    hi_v.at[pl.ds(j * BB, BB)], gsem))
            for h in handles:
                h.wait()

            @plsc.parallel_loop(0, CHUNK // LANES, unroll=2)
            def lerp_group(q):
                lin = iota + q * LANES
                fl = lin >> 7
                bv = lin & (BB - 1)
                wl = wl_v[pl.ds(q * LANES, LANES)]
                wh = wh_v[pl.ds(q * LANES, LANES)]
                for d in range(EMB_DIM):
                    dv = (iota + d) & (EMB_DIM - 1)
                    lo = plsc.load_gather(lo_v, [lin, dv])
                    hi = plsc.load_gather(hi_v, [lin, dv])
                    plsc.store_scatter(
                        out_v, [fl, dv >> 3, dv & 7, bv], lo * wl + hi * wh)

            oh = []
            for fl in range(FG):
                for t in range(DT):
                    oh.append(pltpu.async_copy(
                        out_v.at[fl, t],
                        out_hbm.at[f0 + fl, t, (b0 // BB)], gsem))
            for h in oh:
                h.wait()
            return inner

        lax.fori_loop(0, NFG, fg_body, 0)
        return carry

    lax.fori_loop(0, NBB, bb_body, 0)


@jax.jit
def kernel(x, mask, rand_table, emb):
    table_shift = jnp.int32(N_EMB) * jnp.asarray(rand_table, jnp.int32)
    off = (N_QUANTILES + 2) * jnp.arange(N_FEATURES, dtype=jnp.int32) + table_shift
    off_pad = jnp.zeros((128,), jnp.int32).at[:N_FEATURES].set(off)

    mesh = plsc.VectorSubcoreMesh(core_axis_name="c", subcore_axis_name="s")
    run = pl.kernel(
        _body,
        out_type=jax.ShapeDtypeStruct(
            (N_FEATURES, DT, BATCH // BB, 8, BB), jnp.float32),
        mesh=mesh,
        compiler_params=pltpu.CompilerParams(needs_layout_passes=False,
                                             use_tc_tiling_on_sc=False),
        scratch_types=[
            pltpu.VMEM((BB, N_FEATURES), jnp.float32),
            pltpu.VMEM((BB, N_FEATURES), jnp.int32),
            pltpu.VMEM((128,), jnp.int32),
            pltpu.VMEM((FG, BB), jnp.int32),
            pltpu.VMEM((FG, BB), jnp.int32),
            pltpu.VMEM((CHUNK,), jnp.float32),
            pltpu.VMEM((CHUNK,), jnp.float32),
            pltpu.VMEM((CHUNK, EMB_DIM), jnp.float32),
            pltpu.VMEM((CHUNK, EMB_DIM), jnp.float32),
            pltpu.VMEM((FG, DT, 8, BB), jnp.float32),
            pltpu.SemaphoreType.DMA,
        ],
    )
    out5 = run(x, mask, off_pad, emb)
    out = out5.transpose(0, 1, 3, 2, 4).reshape(N_FEATURES, EMB_DIM, BATCH)
    return out.transpose(2, 0, 1)


# speedup vs baseline: 20.0364x; 1.3217x over previous
import functools

import jax
import jax.numpy as jnp
from jax import lax
from jax.experimental import pallas as pl
from jax.experimental.pallas import tpu as pltpu
from jax.experimental.pallas import tpu_sc as plsc

N_FEATURES = 100
N_QUANTILES = 1000
EMB_DIM = 32
N_EMB = (N_QUANTILES + 2) * N_FEATURES

BATCH = 16384
NW = 32
ROWS_W = BATCH // NW
BB = 128
NBB = ROWS_W // BB
FG = 4
NFG = N_FEATURES // FG
N_CHUNKS = NBB * NFG
CHUNK = FG * BB
LANES = 16
DT = EMB_DIM // 8


def _body(x_hbm, m_hbm, off_hbm, emb_hbm, out_hbm,
          x_v, m_v, off_v,
          ilo0, ilo1, ihi0, ihi1, wl0, wl1, wh0, wh1,
          lo0, lo1, hi0, hi1, ov0, ov1,
          gsem0, gsem1, osem0, osem1):
    wid = lax.axis_index("s") * 2 + lax.axis_index("c")
    th = jnp.float32(1e-06)
    one = jnp.float32(1.0)
    nq = jnp.float32(N_QUANTILES)
    iota = lax.iota(jnp.int32, LANES)

    ilo = (ilo0, ilo1)
    ihi = (ihi0, ihi1)
    wl_ = (wl0, wl1)
    wh_ = (wh0, wh1)
    lo_ = (lo0, lo1)
    hi_ = (hi0, hi1)
    ov = (ov0, ov1)
    gsem = (gsem0, gsem1)
    osem = (osem0, osem1)

    pltpu.sync_copy(off_hbm, off_v)

    def stage_block(c):
        @pl.when(lax.rem(c, jnp.int32(NFG)) == 0)
        def _():
            b0 = wid * ROWS_W + (c // jnp.int32(NFG)) * BB
            pltpu.sync_copy(x_hbm.at[pl.ds(b0, BB)], x_v)
            pltpu.sync_copy(m_hbm.at[pl.ds(b0, BB)], m_v)

    def idx_compute(c, s):
        f0 = lax.rem(c, jnp.int32(NFG)) * FG
        @plsc.parallel_loop(0, CHUNK // LANES, unroll=1)
        def idx_step(i):
            lin = iota + i * LANES
            fl = lin >> 7
            b = lin & (BB - 1)
            f = f0 + fl
            off = plsc.load_gather(off_v, [f])
            xv = plsc.load_gather(x_v, [b, f])
            m = plsc.load_gather(m_v, [b, f])
            xc = jnp.minimum(jnp.maximum(xv, th), one - th)
            y = xc * nq
            yi = y.astype(jnp.int32)
            yh = (y + one).astype(jnp.int32)
            xl = yi.astype(jnp.float32) / nq
            xh = yh.astype(jnp.float32) / nq
            sl = pl.ds(i * LANES, LANES)
            wl_[s][sl] = (xh - xc) * nq
            wh_[s][sl] = (xc - xl) * nq
            plsc.store_scatter(ilo[s], [fl, b], (yi + 1) * m + off)
            plsc.store_scatter(ihi[s], [fl, b], (yh + 1) * m + off)

    def fire_gathers(s):
        for j in range(FG):
            pltpu.async_copy(emb_hbm.at[ilo[s].at[j]],
                             lo_[s].at[pl.ds(j * BB, BB)], gsem[s])
            pltpu.async_copy(emb_hbm.at[ihi[s].at[j]],
                             hi_[s].at[pl.ds(j * BB, BB)], gsem[s])

    def wait_gathers(s):
        for j in range(FG):
            pltpu.make_async_copy(emb_hbm.at[ilo[s].at[j]],
                                  lo_[s].at[pl.ds(j * BB, BB)], gsem[s]).wait()
            pltpu.make_async_copy(emb_hbm.at[ihi[s].at[j]],
                                  hi_[s].at[pl.ds(j * BB, BB)], gsem[s]).wait()

    def lerp(c, s):
        @plsc.parallel_loop(0, CHUNK // LANES, unroll=1)
        def lerp_group(q):
            lin = iota + q * LANES
            fl = lin >> 7
            bv = lin & (BB - 1)
            wl = wl_[s][pl.ds(q * LANES, LANES)]
            wh = wh_[s][pl.ds(q * LANES, LANES)]
            for d in range(EMB_DIM):
                dv = (iota + d) & (EMB_DIM - 1)
                lo = plsc.load_gather(lo_[s], [lin, dv])
                hi = plsc.load_gather(hi_[s], [lin, dv])
                plsc.store_scatter(
                    ov[s], [fl, dv >> 3, dv & 7, bv], lo * wl + hi * wh)

    def fire_out(c, s):
        f0 = lax.rem(c, jnp.int32(NFG)) * FG
        bb = c // jnp.int32(NFG) + wid * NBB
        for fl in range(FG):
            for t in range(DT):
                pltpu.async_copy(ov[s].at[fl, t],
                                 out_hbm.at[f0 + fl, t, bb], osem[s])

    def wait_out(c, s):
        f0 = lax.rem(c, jnp.int32(NFG)) * FG
        bb = c // jnp.int32(NFG) + wid * NBB
        for fl in range(FG):
            for t in range(DT):
                pltpu.make_async_copy(ov[s].at[fl, t],
                                      out_hbm.at[f0 + fl, t, bb],
                                      osem[s]).wait()

    stage_block(jnp.int32(0))
    idx_compute(jnp.int32(0), 0)
    fire_gathers(0)

    def pair_body(p, carry):
        for par in (0, 1):
            c = 2 * p + par
            s = par
            @pl.when(c < N_CHUNKS - 1)
            def _():
                stage_block(c + 1)
                idx_compute(c + 1, 1 - s)

                @pl.when(c >= 1)
                def _():
                    wait_out(c - 1, 1 - s)
                fire_gathers(1 - s)
            wait_gathers(s)
            lerp(c, s)
            fire_out(c, s)
        return carry

    lax.fori_loop(0, N_CHUNKS // 2, pair_body, 0)
    wait_out(jnp.int32(N_CHUNKS - 2), 0)
    wait_out(jnp.int32(N_CHUNKS - 1), 1)


@jax.jit
def kernel(x, mask, rand_table, emb):
    table_shift = jnp.int32(N_EMB) * jnp.asarray(rand_table, jnp.int32)
    off = (N_QUANTILES + 2) * jnp.arange(N_FEATURES, dtype=jnp.int32) + table_shift
    off_pad = jnp.zeros((128,), jnp.int32).at[:N_FEATURES].set(off)

    mesh = plsc.VectorSubcoreMesh(core_axis_name="c", subcore_axis_name="s")
    run = pl.kernel(
        _body,
        out_type=jax.ShapeDtypeStruct(
            (N_FEATURES, DT, BATCH // BB, 8, BB), jnp.float32),
        mesh=mesh,
        compiler_params=pltpu.CompilerParams(needs_layout_passes=False,
                                             use_tc_tiling_on_sc=False),
        scratch_types=[
            pltpu.VMEM((BB, N_FEATURES), jnp.float32),
            pltpu.VMEM((BB, N_FEATURES), jnp.int32),
            pltpu.VMEM((128,), jnp.int32),
            pltpu.VMEM((FG, BB), jnp.int32),
            pltpu.VMEM((FG, BB), jnp.int32),
            pltpu.VMEM((FG, BB), jnp.int32),
            pltpu.VMEM((FG, BB), jnp.int32),
            pltpu.VMEM((CHUNK,), jnp.float32),
            pltpu.VMEM((CHUNK,), jnp.float32),
            pltpu.VMEM((CHUNK,), jnp.float32),
            pltpu.VMEM((CHUNK,), jnp.float32),
            pltpu.VMEM((CHUNK, EMB_DIM), jnp.float32),
            pltpu.VMEM((CHUNK, EMB_DIM), jnp.float32),
            pltpu.VMEM((CHUNK, EMB_DIM), jnp.float32),
            pltpu.VMEM((CHUNK, EMB_DIM), jnp.float32),
            pltpu.VMEM((FG, DT, 8, BB), jnp.float32),
            pltpu.VMEM((FG, DT, 8, BB), jnp.float32),
            pltpu.SemaphoreType.DMA,
            pltpu.SemaphoreType.DMA,
            pltpu.SemaphoreType.DMA,
            pltpu.SemaphoreType.DMA,
        ],
    )
    out5 = run(x, mask, off_pad, emb)
    out = out5.transpose(0, 1, 3, 2, 4).reshape(N_FEATURES, EMB_DIM, BATCH)
    return out.transpose(2, 0, 1)
